# Initial kernel scaffold; baseline (speedup 1.0000x reference)
import functools

import jax
import jax.numpy as jnp
from jax import lax
from jax.experimental import pallas as pl
from jax.experimental.pallas import tpu as pltpu
from jax.experimental.pallas import tpu_sc as plsc

LATENT = 128
N_NODES = 10000
N_EDGES = 320000
N_GRAPHS = 64
MAX_LV = 3

NC, NS = 2, 16
NW = NC * NS
E_PER_W = N_EDGES // NW
CHUNK = 80
N_CHUNKS = E_PER_W // CHUNK
N_PAIRS = (N_CHUNKS - 1) // 2
ROWS_PER_SUB = N_NODES // NS
ZROWS = 25

BLK = 1000
N_BLK = N_NODES // BLK


def _spmm_body(cur_hbm, src_hbm, dst_hbm, agg_hbm,
               agg_sh, src_v, dst_v, rows0, rows1, zbuf, sem0, sem1):
    c = lax.axis_index("c")
    s = lax.axis_index("s")
    wid = c * NS + s

    pltpu.sync_copy(src_hbm.at[wid], src_v)
    pltpu.sync_copy(dst_hbm.at[wid], dst_v)

    zero = jnp.zeros((16,), jnp.float32)
    for r in range(ZROWS):
        for q in range(8):
            zbuf[r, pl.ds(q * 16, 16)] = zero
    base_row = s * ROWS_PER_SUB

    def zbody(i, carry):
        pltpu.sync_copy(zbuf, agg_sh.at[pl.ds(base_row + i * ZROWS, ZROWS)])
        return carry

    lax.fori_loop(0, ROWS_PER_SUB // ZROWS, zbody, 0)
    plsc.subcore_barrier()

    pltpu.async_copy(cur_hbm.at[src_v.at[0]], rows0, sem0)

    def body(t, carry):
        j0 = 2 * t
        d1 = pltpu.async_copy(cur_hbm.at[src_v.at[j0 + 1]], rows1, sem1)
        pltpu.make_async_copy(cur_hbm.at[src_v.at[j0]], rows0, sem0).wait()
        pltpu.sync_copy(rows0, agg_sh.at[dst_v.at[j0]], add=True)
        pltpu.async_copy(cur_hbm.at[src_v.at[j0 + 2]], rows0, sem0)
        d1.wait()
        pltpu.sync_copy(rows1, agg_sh.at[dst_v.at[j0 + 1]], add=True)
        return carry

    lax.fori_loop(0, N_PAIRS, body, 0)
    last = N_CHUNKS - 1
    pltpu.make_async_copy(cur_hbm.at[src_v.at[last]], rows0, sem0).wait()
    pltpu.sync_copy(rows0, agg_sh.at[dst_v.at[last]], add=True)

    plsc.subcore_barrier()
    pltpu.sync_copy(agg_sh.at[pl.ds(base_row, ROWS_PER_SUB)],
                    agg_hbm.at[c, pl.ds(base_row, ROWS_PER_SUB)])


_spmm = pl.kernel(
    _spmm_body,
    out_type=jax.ShapeDtypeStruct((NC, N_NODES, LATENT), jnp.float32),
    mesh=plsc.VectorSubcoreMesh(core_axis_name="c", subcore_axis_name="s",
                                num_cores=NC, num_subcores=NS),
    scratch_types=[
        pltpu.VMEM_SHARED((N_NODES, LATENT), jnp.float32),
        pltpu.VMEM((N_CHUNKS, CHUNK), jnp.int32),
        pltpu.VMEM((N_CHUNKS, CHUNK), jnp.int32),
        pltpu.VMEM((CHUNK, LATENT), jnp.float32),
        pltpu.VMEM((CHUNK, LATENT), jnp.float32),
        pltpu.VMEM((ZROWS, LATENT), jnp.float32),
        pltpu.SemaphoreType.DMA,
        pltpu.SemaphoreType.DMA,
    ],
)


def _proj_body(x_ref, w_ref, im_ref, cur_ref):
    im = jnp.dot(x_ref[...], w_ref[...], preferred_element_type=jnp.float32)
    im_ref[...] = im
    cur_ref[...] = jnp.maximum(im, 0.0)


def _proj(x, w):
    return pl.pallas_call(
        _proj_body,
        grid=(N_BLK,),
        in_specs=[
            pl.BlockSpec((BLK, LATENT), lambda i: (i, 0)),
            pl.BlockSpec((LATENT, LATENT), lambda i: (0, 0)),
        ],
        out_specs=[
            pl.BlockSpec((BLK, LATENT), lambda i: (i, 0)),
            pl.BlockSpec((BLK, LATENT), lambda i: (i, 0)),
        ],
        out_shape=[
            jax.ShapeDtypeStruct((N_NODES, LATENT), jnp.float32),
            jax.ShapeDtypeStruct((N_NODES, LATENT), jnp.float32),
        ],
    )(x, w)


def _upd_body(agg_ref, im_ref, w_ref, cur_ref):
    aggs = agg_ref[0] + agg_ref[1]
    h = jnp.dot(aggs, w_ref[...], preferred_element_type=jnp.float32)
    cur_ref[...] = jnp.maximum(h + im_ref[...], 0.0)


def _upd(agg, im, w):
    return pl.pallas_call(
        _upd_body,
        grid=(N_BLK,),
        in_specs=[
            pl.BlockSpec((NC, BLK, LATENT), lambda i: (0, i, 0)),
            pl.BlockSpec((BLK, LATENT), lambda i: (i, 0)),
            pl.BlockSpec((LATENT, LATENT), lambda i: (0, 0)),
        ],
        out_specs=pl.BlockSpec((BLK, LATENT), lambda i: (i, 0)),
        out_shape=jax.ShapeDtypeStruct((N_NODES, LATENT), jnp.float32),
    )(agg, im, w)


def _fin_body(agg_ref, im_ref, gid_ref, wc_ref, wo_ref, out_ref, pooled):
    i = pl.program_id(0)

    @pl.when(i == 0)
    def _():
        pooled[...] = jnp.zeros_like(pooled)

    aggs = agg_ref[0] + agg_ref[1]
    h = jnp.dot(aggs, wc_ref[...], preferred_element_type=jnp.float32)
    cur = jnp.maximum(h + im_ref[...], 0.0)
    ids = gid_ref[0, 0, :]
    onehot = (ids[None, :] == lax.broadcasted_iota(jnp.int32, (N_GRAPHS, BLK), 0)
              ).astype(jnp.float32)
    pooled[...] += jnp.dot(onehot, cur, preferred_element_type=jnp.float32)

    @pl.when(i == N_BLK - 1)
    def _():
        out_ref[...] = jnp.maximum(
            jnp.dot(pooled[...], wo_ref[...], preferred_element_type=jnp.float32), 0.0)


def _fin(agg, im, gid, wc, wo):
    return pl.pallas_call(
        _fin_body,
        grid=(N_BLK,),
        in_specs=[
            pl.BlockSpec((NC, BLK, LATENT), lambda i: (0, i, 0)),
            pl.BlockSpec((BLK, LATENT), lambda i: (i, 0)),
            pl.BlockSpec((1, 1, BLK), lambda i: (i, 0, 0)),
            pl.BlockSpec((LATENT, LATENT), lambda i: (0, 0)),
            pl.BlockSpec((LATENT, LATENT), lambda i: (0, 0)),
        ],
        out_specs=pl.BlockSpec((N_GRAPHS, LATENT), lambda i: (0, 0)),
        out_shape=jax.ShapeDtypeStruct((N_GRAPHS, LATENT), jnp.float32),
        scratch_shapes=[pltpu.VMEM((N_GRAPHS, LATENT), jnp.float32)],
    )(agg, im, gid, wc, wo)


@jax.jit
def _run(x, edge_index, graph_ids, W_n2l, W_conv, W_out):
    src = edge_index[0].astype(jnp.int32).reshape(NW, N_CHUNKS, CHUNK)
    dst = edge_index[1].astype(jnp.int32).reshape(NW, N_CHUNKS, CHUNK)
    gid = graph_ids.astype(jnp.int32).reshape(N_BLK, 1, BLK)
    im, cur = _proj(x, W_n2l)
    for _ in range(MAX_LV - 1):
        agg = _spmm(cur, src, dst)
        cur = _upd(agg, im, W_conv)
    agg = _spmm(cur, src, dst)
    return _fin(agg, im, gid, W_conv, W_out)


def kernel(x, edge_index, graph_ids, W_n2l, W_conv, W_out):
    return _run(x, edge_index, graph_ids, W_n2l, W_conv, W_out)


# speedup vs baseline: 11.9790x; 11.9790x over previous
import functools

import jax
import jax.numpy as jnp
from jax import lax
from jax.experimental import pallas as pl
from jax.experimental.pallas import tpu as pltpu
from jax.experimental.pallas import tpu_sc as plsc

LATENT = 128
N_NODES = 10000
N_EDGES = 320000
N_GRAPHS = 64
MAX_LV = 3

NC, NS = 2, 16
NW = NC * NS
E_PER_W = N_EDGES // NW
CHUNK = 80
N_CHUNKS = E_PER_W // CHUNK
N_PAIRS = (N_CHUNKS - 1) // 2
ROWS_PER_SUB = N_NODES // NS
ZROWS = 25

BLK = 1000
N_BLK = N_NODES // BLK


def _spmm_body(cur_hbm, src_hbm, dst_hbm, agg_hbm,
               agg_sh, src_v, dst_v, rows0, rows1, zbuf, sem0, sem1):
    c = lax.axis_index("c")
    s = lax.axis_index("s")
    wid = c * NS + s

    pltpu.sync_copy(src_hbm.at[wid], src_v)
    pltpu.sync_copy(dst_hbm.at[wid], dst_v)

    zero = jnp.zeros((16,), jnp.float32)
    for r in range(ZROWS):
        for q in range(8):
            zbuf[r, pl.ds(q * 16, 16)] = zero
    base_row = s * ROWS_PER_SUB

    def zbody(i, carry):
        pltpu.sync_copy(zbuf, agg_sh.at[pl.ds(base_row + i * ZROWS, ZROWS)])
        return carry

    lax.fori_loop(0, ROWS_PER_SUB // ZROWS, zbody, 0)
    plsc.subcore_barrier()

    pltpu.async_copy(cur_hbm.at[src_v.at[0]], rows0, sem0)

    def body(t, carry):
        j0 = 2 * t
        d1 = pltpu.async_copy(cur_hbm.at[src_v.at[j0 + 1]], rows1, sem1)
        pltpu.make_async_copy(cur_hbm.at[src_v.at[j0]], rows0, sem0).wait()
        pltpu.sync_copy(rows0, agg_sh.at[dst_v.at[j0]], add=True)
        pltpu.async_copy(cur_hbm.at[src_v.at[j0 + 2]], rows0, sem0)
        d1.wait()
        pltpu.sync_copy(rows1, agg_sh.at[dst_v.at[j0 + 1]], add=True)
        return carry

    lax.fori_loop(0, N_PAIRS, body, 0)
    last = N_CHUNKS - 1
    pltpu.make_async_copy(cur_hbm.at[src_v.at[last]], rows0, sem0).wait()
    pltpu.sync_copy(rows0, agg_sh.at[dst_v.at[last]], add=True)

    plsc.subcore_barrier()
    pltpu.sync_copy(agg_sh.at[pl.ds(base_row, ROWS_PER_SUB)],
                    agg_hbm.at[c, pl.ds(base_row, ROWS_PER_SUB)])


@functools.lru_cache(maxsize=1)
def _get_spmm():
    return pl.kernel(
        _spmm_body,
        out_type=jax.ShapeDtypeStruct((NC, N_NODES, LATENT), jnp.float32),
        mesh=plsc.VectorSubcoreMesh(core_axis_name="c", subcore_axis_name="s",
                                    num_cores=NC, num_subcores=NS),
        scratch_types=[
            pltpu.VMEM_SHARED((N_NODES, LATENT), jnp.float32),
            pltpu.VMEM((N_CHUNKS, CHUNK), jnp.int32),
            pltpu.VMEM((N_CHUNKS, CHUNK), jnp.int32),
            pltpu.VMEM((CHUNK, LATENT), jnp.float32),
            pltpu.VMEM((CHUNK, LATENT), jnp.float32),
            pltpu.VMEM((ZROWS, LATENT), jnp.float32),
            pltpu.SemaphoreType.DMA,
            pltpu.SemaphoreType.DMA,
        ],
        compiler_params=pltpu.CompilerParams(use_tc_tiling_on_sc=False),
    )


def _spmm(cur, src, dst):
    return _get_spmm()(cur, src, dst)


def _proj_body(x_ref, w_ref, im_ref, cur_ref):
    im = jnp.dot(x_ref[...], w_ref[...], preferred_element_type=jnp.float32)
    im_ref[...] = im
    cur_ref[...] = jnp.maximum(im, 0.0)


def _proj(x, w):
    return pl.pallas_call(
        _proj_body,
        grid=(N_BLK,),
        in_specs=[
            pl.BlockSpec((BLK, LATENT), lambda i: (i, 0)),
            pl.BlockSpec((LATENT, LATENT), lambda i: (0, 0)),
        ],
        out_specs=[
            pl.BlockSpec((BLK, LATENT), lambda i: (i, 0)),
            pl.BlockSpec((BLK, LATENT), lambda i: (i, 0)),
        ],
        out_shape=[
            jax.ShapeDtypeStruct((N_NODES, LATENT), jnp.float32),
            jax.ShapeDtypeStruct((N_NODES, LATENT), jnp.float32),
        ],
    )(x, w)


def _upd_body(agg_ref, im_ref, w_ref, cur_ref):
    aggs = agg_ref[0] + agg_ref[1]
    h = jnp.dot(aggs, w_ref[...], preferred_element_type=jnp.float32)
    cur_ref[...] = jnp.maximum(h + im_ref[...], 0.0)


def _upd(agg, im, w):
    return pl.pallas_call(
        _upd_body,
        grid=(N_BLK,),
        in_specs=[
            pl.BlockSpec((NC, BLK, LATENT), lambda i: (0, i, 0)),
            pl.BlockSpec((BLK, LATENT), lambda i: (i, 0)),
            pl.BlockSpec((LATENT, LATENT), lambda i: (0, 0)),
        ],
        out_specs=pl.BlockSpec((BLK, LATENT), lambda i: (i, 0)),
        out_shape=jax.ShapeDtypeStruct((N_NODES, LATENT), jnp.float32),
    )(agg, im, w)


def _fin_body(agg_ref, im_ref, gid_ref, wc_ref, wo_ref, out_ref, pooled):
    i = pl.program_id(0)

    @pl.when(i == 0)
    def _():
        pooled[...] = jnp.zeros_like(pooled)

    aggs = agg_ref[0] + agg_ref[1]
    h = jnp.dot(aggs, wc_ref[...], preferred_element_type=jnp.float32)
    cur = jnp.maximum(h + im_ref[...], 0.0)
    ids = gid_ref[0, 0, :]
    onehot = (ids[None, :] == lax.broadcasted_iota(jnp.int32, (N_GRAPHS, BLK), 0)
              ).astype(jnp.float32)
    pooled[...] += jnp.dot(onehot, cur, preferred_element_type=jnp.float32)

    @pl.when(i == N_BLK - 1)
    def _():
        out_ref[...] = jnp.maximum(
            jnp.dot(pooled[...], wo_ref[...], preferred_element_type=jnp.float32), 0.0)


def _fin(agg, im, gid, wc, wo):
    return pl.pallas_call(
        _fin_body,
        grid=(N_BLK,),
        in_specs=[
            pl.BlockSpec((NC, BLK, LATENT), lambda i: (0, i, 0)),
            pl.BlockSpec((BLK, LATENT), lambda i: (i, 0)),
            pl.BlockSpec((1, 1, BLK), lambda i: (i, 0, 0)),
            pl.BlockSpec((LATENT, LATENT), lambda i: (0, 0)),
            pl.BlockSpec((LATENT, LATENT), lambda i: (0, 0)),
        ],
        out_specs=pl.BlockSpec((N_GRAPHS, LATENT), lambda i: (0, 0)),
        out_shape=jax.ShapeDtypeStruct((N_GRAPHS, LATENT), jnp.float32),
        scratch_shapes=[pltpu.VMEM((N_GRAPHS, LATENT), jnp.float32)],
    )(agg, im, gid, wc, wo)


@jax.jit
def _run(x, edge_index, graph_ids, W_n2l, W_conv, W_out):
    src = edge_index[0].astype(jnp.int32).reshape(NW, N_CHUNKS, CHUNK)
    dst = edge_index[1].astype(jnp.int32).reshape(NW, N_CHUNKS, CHUNK)
    gid = graph_ids.astype(jnp.int32).reshape(N_BLK, 1, BLK)
    im, cur = _proj(x, W_n2l)
    for _ in range(MAX_LV - 1):
        agg = _spmm(cur, src, dst)
        cur = _upd(agg, im, W_conv)
    agg = _spmm(cur, src, dst)
    return _fin(agg, im, gid, W_conv, W_out)


def kernel(x, edge_index, graph_ids, W_n2l, W_conv, W_out):
    return _run(x, edge_index, graph_ids, W_n2l, W_conv, W_out)


# speedup vs baseline: 12.4806x; 1.0419x over previous
import functools

import jax
import jax.numpy as jnp
from jax import lax
from jax.experimental import pallas as pl
from jax.experimental.pallas import tpu as pltpu
from jax.experimental.pallas import tpu_sc as plsc

LATENT = 128
N_NODES = 10000
N_EDGES = 320000
N_GRAPHS = 64
MAX_LV = 3

NC, NS = 2, 16
NW = NC * NS
E_PER_W = N_EDGES // NW
CHUNK = 100
N_CHUNKS = E_PER_W // CHUNK
N_PAIRS = N_CHUNKS // 2
ROWS_PER_SUB = N_NODES // NS
ZROWS = 25

BLK = 1000
N_BLK = N_NODES // BLK


def _spmm_body(cur_hbm, src_hbm, dst_hbm, agg_hbm,
               agg_sh, src_v, dst_v, rows0, rows1, zbuf, sem0, sem1):
    c = lax.axis_index("c")
    s = lax.axis_index("s")
    wid = c * NS + s

    pltpu.sync_copy(src_hbm.at[wid], src_v)
    pltpu.sync_copy(dst_hbm.at[wid], dst_v)

    zero = jnp.zeros((16,), jnp.float32)
    for r in range(ZROWS):
        for q in range(8):
            zbuf[r, pl.ds(q * 16, 16)] = zero
    base_row = s * ROWS_PER_SUB

    def zbody(i, carry):
        pltpu.sync_copy(zbuf, agg_sh.at[pl.ds(base_row + i * ZROWS, ZROWS)])
        return carry

    lax.fori_loop(0, ROWS_PER_SUB // ZROWS, zbody, 0)
    plsc.subcore_barrier()

    pltpu.async_copy(cur_hbm.at[src_v.at[0]], rows0, sem0)

    def body(t, carry):
        j0 = 2 * t
        d1 = pltpu.async_copy(cur_hbm.at[src_v.at[j0 + 1]], rows1, sem1)
        pltpu.make_async_copy(cur_hbm.at[src_v.at[j0]], rows0, sem0).wait()
        pltpu.sync_copy(rows0, agg_sh.at[dst_v.at[j0]], add=True)
        pltpu.async_copy(cur_hbm.at[src_v.at[j0 + 2]], rows0, sem0)
        d1.wait()
        pltpu.sync_copy(rows1, agg_sh.at[dst_v.at[j0 + 1]], add=True)
        return carry

    lax.fori_loop(0, N_PAIRS - 1, body, 0)
    j0 = N_CHUNKS - 2
    d1 = pltpu.async_copy(cur_hbm.at[src_v.at[j0 + 1]], rows1, sem1)
    pltpu.make_async_copy(cur_hbm.at[src_v.at[j0]], rows0, sem0).wait()
    pltpu.sync_copy(rows0, agg_sh.at[dst_v.at[j0]], add=True)
    d1.wait()
    pltpu.sync_copy(rows1, agg_sh.at[dst_v.at[j0 + 1]], add=True)

    plsc.subcore_barrier()
    pltpu.sync_copy(agg_sh.at[pl.ds(base_row, ROWS_PER_SUB)],
                    agg_hbm.at[c, pl.ds(base_row, ROWS_PER_SUB)])


@functools.lru_cache(maxsize=1)
def _get_spmm():
    return pl.kernel(
        _spmm_body,
        out_type=jax.ShapeDtypeStruct((NC, N_NODES, LATENT), jnp.float32),
        mesh=plsc.VectorSubcoreMesh(core_axis_name="c", subcore_axis_name="s",
                                    num_cores=NC, num_subcores=NS),
        scratch_types=[
            pltpu.VMEM_SHARED((N_NODES, LATENT), jnp.float32),
            pltpu.VMEM((N_CHUNKS, CHUNK), jnp.int32),
            pltpu.VMEM((N_CHUNKS, CHUNK), jnp.int32),
            pltpu.VMEM((CHUNK, LATENT), jnp.float32),
            pltpu.VMEM((CHUNK, LATENT), jnp.float32),
            pltpu.VMEM((ZROWS, LATENT), jnp.float32),
            pltpu.SemaphoreType.DMA,
            pltpu.SemaphoreType.DMA,
        ],
        compiler_params=pltpu.CompilerParams(use_tc_tiling_on_sc=False),
    )


def _spmm(cur, src, dst):
    return _get_spmm()(cur, src, dst)


def _proj_body(x_ref, w_ref, im_ref, cur_ref):
    im = jnp.dot(x_ref[...], w_ref[...], preferred_element_type=jnp.float32)
    im_ref[...] = im
    cur_ref[...] = jnp.maximum(im, 0.0)


def _proj(x, w):
    return pl.pallas_call(
        _proj_body,
        grid=(N_BLK,),
        in_specs=[
            pl.BlockSpec((BLK, LATENT), lambda i: (i, 0)),
            pl.BlockSpec((LATENT, LATENT), lambda i: (0, 0)),
        ],
        out_specs=[
            pl.BlockSpec((BLK, LATENT), lambda i: (i, 0)),
            pl.BlockSpec((BLK, LATENT), lambda i: (i, 0)),
        ],
        out_shape=[
            jax.ShapeDtypeStruct((N_NODES, LATENT), jnp.float32),
            jax.ShapeDtypeStruct((N_NODES, LATENT), jnp.float32),
        ],
    )(x, w)


def _upd_body(agg_ref, im_ref, w_ref, cur_ref):
    aggs = agg_ref[0] + agg_ref[1]
    h = jnp.dot(aggs, w_ref[...], preferred_element_type=jnp.float32)
    cur_ref[...] = jnp.maximum(h + im_ref[...], 0.0)


def _upd(agg, im, w):
    return pl.pallas_call(
        _upd_body,
        grid=(N_BLK,),
        in_specs=[
            pl.BlockSpec((NC, BLK, LATENT), lambda i: (0, i, 0)),
            pl.BlockSpec((BLK, LATENT), lambda i: (i, 0)),
            pl.BlockSpec((LATENT, LATENT), lambda i: (0, 0)),
        ],
        out_specs=pl.BlockSpec((BLK, LATENT), lambda i: (i, 0)),
        out_shape=jax.ShapeDtypeStruct((N_NODES, LATENT), jnp.float32),
    )(agg, im, w)


def _fin_body(agg_ref, im_ref, gid_ref, wc_ref, wo_ref, out_ref, pooled):
    i = pl.program_id(0)

    @pl.when(i == 0)
    def _():
        pooled[...] = jnp.zeros_like(pooled)

    aggs = agg_ref[0] + agg_ref[1]
    h = jnp.dot(aggs, wc_ref[...], preferred_element_type=jnp.float32)
    cur = jnp.maximum(h + im_ref[...], 0.0)
    ids = gid_ref[0, 0, :]
    onehot = (ids[None, :] == lax.broadcasted_iota(jnp.int32, (N_GRAPHS, BLK), 0)
              ).astype(jnp.float32)
    pooled[...] += jnp.dot(onehot, cur, preferred_element_type=jnp.float32)

    @pl.when(i == N_BLK - 1)
    def _():
        out_ref[...] = jnp.maximum(
            jnp.dot(pooled[...], wo_ref[...], preferred_element_type=jnp.float32), 0.0)


def _fin(agg, im, gid, wc, wo):
    return pl.pallas_call(
        _fin_body,
        grid=(N_BLK,),
        in_specs=[
            pl.BlockSpec((NC, BLK, LATENT), lambda i: (0, i, 0)),
            pl.BlockSpec((BLK, LATENT), lambda i: (i, 0)),
            pl.BlockSpec((1, 1, BLK), lambda i: (i, 0, 0)),
            pl.BlockSpec((LATENT, LATENT), lambda i: (0, 0)),
            pl.BlockSpec((LATENT, LATENT), lambda i: (0, 0)),
        ],
        out_specs=pl.BlockSpec((N_GRAPHS, LATENT), lambda i: (0, 0)),
        out_shape=jax.ShapeDtypeStruct((N_GRAPHS, LATENT), jnp.float32),
        scratch_shapes=[pltpu.VMEM((N_GRAPHS, LATENT), jnp.float32)],
    )(agg, im, gid, wc, wo)


@jax.jit
def _run(x, edge_index, graph_ids, W_n2l, W_conv, W_out):
    src = edge_index[0].astype(jnp.int32).reshape(NW, N_CHUNKS, CHUNK)
    dst = edge_index[1].astype(jnp.int32).reshape(NW, N_CHUNKS, CHUNK)
    gid = graph_ids.astype(jnp.int32).reshape(N_BLK, 1, BLK)
    im, cur = _proj(x, W_n2l)
    for _ in range(MAX_LV - 1):
        agg = _spmm(cur, src, dst)
        cur = _upd(agg, im, W_conv)
    agg = _spmm(cur, src, dst)
    return _fin(agg, im, gid, W_conv, W_out)


def kernel(x, edge_index, graph_ids, W_n2l, W_conv, W_out):
    return _run(x, edge_index, graph_ids, W_n2l, W_conv, W_out)
